# baseline (speedup 1.0000x reference)
import functools

import jax
import jax.numpy as jnp
from jax import lax
from jax.experimental import pallas as pl
from jax.experimental.pallas import tpu as pltpu
from jax.experimental.pallas import tpu_sc as plsc

N = 10000
E = 320000
D_IN = 128
HID = 16
FS = 4

NC = 2
NS = 16
NW = NC * NS
L = 16

EPT = 10240
E_PAD = NW * EPT

_f32 = jnp.float32
_i32 = jnp.int32


def _t_matmul(w_ref, x_ref):
  return lax.dot_general(w_ref[:], x_ref[:], (((0,), (0,)), ((), ())),
                         preferred_element_type=_f32)


def _dense_in_body(h_ref, w_ref, asrc_ref, adst_ref, xw_out, as_out, ad_out):
  xw_t = lax.dot_general(w_ref[:], h_ref[:], (((0,), (1,)), ((), ())),
                         preferred_element_type=_f32)
  xw_out[:] = xw_t
  as_out[:] = jnp.dot(asrc_ref[:], xw_t, preferred_element_type=_f32)
  ad_out[:] = jnp.dot(adst_ref[:], xw_t, preferred_element_type=_f32)


def _dense_in(h, w, asrc, adst):
  return pl.pallas_call(
      _dense_in_body,
      out_shape=[
          jax.ShapeDtypeStruct((HID, N), _f32),
          jax.ShapeDtypeStruct((1, N), _f32),
          jax.ShapeDtypeStruct((1, N), _f32),
      ],
  )(h, w, asrc, adst)


def _combine(nump_ref, denp_ref, b_ref):
  num = jnp.sum(nump_ref[:], axis=1)
  den = jnp.sum(denp_ref[:], axis=0)
  return jnp.maximum(num / (den[None, :] + 1e-16) + b_ref[:], 0.0)


def _dense_mid_body(nump_ref, denp_ref, b_ref, w_ref, asrc_ref, adst_ref,
                    xw_out, as_out, ad_out):
  x_t = _combine(nump_ref, denp_ref, b_ref)
  xw_t = _t_matmul(w_ref, x_t)
  xw_out[:] = xw_t
  as_out[:] = jnp.dot(asrc_ref[:], xw_t, preferred_element_type=_f32)
  ad_out[:] = jnp.dot(adst_ref[:], xw_t, preferred_element_type=_f32)


def _dense_mid(nump, denp, b, w, asrc, adst):
  return pl.pallas_call(
      _dense_mid_body,
      out_shape=[
          jax.ShapeDtypeStruct((HID, N), _f32),
          jax.ShapeDtypeStruct((1, N), _f32),
          jax.ShapeDtypeStruct((1, N), _f32),
      ],
  )(nump, denp, b, w, asrc, adst)


def _dense_out_body(nump_ref, denp_ref, b_ref, w_ref, bout_ref, out_ref):
  x_t = _combine(nump_ref, denp_ref, b_ref)
  out_ref[:] = _t_matmul(w_ref, x_t) + bout_ref[:]


def _dense_out(nump, denp, b, w, bout):
  return pl.pallas_call(
      _dense_out_body,
      out_shape=jax.ShapeDtypeStruct((1, N), _f32),
  )(nump, denp, b, w, bout)


def _attn_body(src_hbm, dst_hbm, asrc_hbm, adst_hbm,
               ex_out, den_out,
               src_v, dst_v, asrc_v, adst_v, den_v, ex_v):
  cid = lax.axis_index("c")
  sid = lax.axis_index("s")
  wid = sid * NC + cid
  e0 = pl.multiple_of(wid * EPT, 8)

  pltpu.sync_copy(src_hbm.at[pl.ds(e0, EPT)], src_v)
  pltpu.sync_copy(dst_hbm.at[pl.ds(e0, EPT)], dst_v)
  pltpu.sync_copy(asrc_hbm, asrc_v)
  pltpu.sync_copy(adst_hbm, adst_v)

  zero16 = jnp.zeros((L,), _f32)

  def _zden(i, carry):
    den_v[pl.ds(i * L, L)] = zero16
    return carry
  lax.fori_loop(0, N // L, _zden, 0)

  iota = lax.iota(_i32, L)
  ebase = wid * EPT

  def _chunk(j, carry):
    s_vec = src_v[pl.ds(j * L, L)]
    d_vec = dst_v[pl.ds(j * L, L)]
    a = plsc.load_gather(asrc_v, [s_vec]) + plsc.load_gather(adst_v, [d_vec])
    a = jnp.maximum(a, a * 0.2)
    ex = jnp.exp(a)
    ex = jnp.where(ebase + j * L + iota < E, ex, 0.0)
    plsc.addupdate_scatter(den_v, [d_vec], ex)
    ex_v[pl.ds(j * L, L)] = ex
    return carry
  lax.fori_loop(0, EPT // L, _chunk, 0, unroll=4)

  pltpu.sync_copy(ex_v, ex_out.at[pl.ds(e0, EPT)])
  d0 = pl.multiple_of(wid * N, 8)
  pltpu.sync_copy(den_v, den_out.at[pl.ds(d0, N)])


def _attn_phase(src_f, dst_f, alpha_src, alpha_dst):
  mesh = plsc.VectorSubcoreMesh(core_axis_name="c", subcore_axis_name="s")
  kernel_fn = functools.partial(
      pl.kernel,
      out_type=[
          jax.ShapeDtypeStruct((E_PAD,), _f32),
          jax.ShapeDtypeStruct((NW * N,), _f32),
      ],
      mesh=mesh,
      scratch_types=[
          pltpu.VMEM((EPT,), _i32),
          pltpu.VMEM((EPT,), _i32),
          pltpu.VMEM((N,), _f32),
          pltpu.VMEM((N,), _f32),
          pltpu.VMEM((N,), _f32),
          pltpu.VMEM((EPT,), _f32),
      ],
      compiler_params=pltpu.CompilerParams(needs_layout_passes=False,
                                           use_tc_tiling_on_sc=False),
  )(_attn_body)
  return kernel_fn(src_f, dst_f, alpha_src, alpha_dst)


def _msg_body(src_hbm, dst_hbm, ex_hbm, xwt_hbm,
              num_out,
              src_v, dst_v, ex_v, xw_v, acc_v):
  cid = lax.axis_index("c")
  sid = lax.axis_index("s")
  wid = sid * NC + cid
  e0 = pl.multiple_of(wid * EPT, 8)

  pltpu.sync_copy(src_hbm.at[pl.ds(e0, EPT)], src_v)
  pltpu.sync_copy(dst_hbm.at[pl.ds(e0, EPT)], dst_v)
  pltpu.sync_copy(ex_hbm.at[pl.ds(e0, EPT)], ex_v)

  zero16 = jnp.zeros((L,), _f32)
  n0 = pl.multiple_of(wid * N, 8)

  for s in range(HID // FS):
    pltpu.sync_copy(xwt_hbm.at[pl.ds(s * FS, FS)], xw_v)

    def _zacc(i, carry):
      acc_v[i % FS, pl.ds((i // FS) * L, L)] = zero16
      return carry
    lax.fori_loop(0, FS * (N // L), _zacc, 0)

    def _chunk(j, carry):
      s_vec = src_v[pl.ds(j * L, L)]
      d_vec = dst_v[pl.ds(j * L, L)]
      exv = ex_v[pl.ds(j * L, L)]
      for k in range(FS):
        bk = jnp.full((L,), k, _i32)
        xg = plsc.load_gather(xw_v, [bk, s_vec])
        plsc.addupdate_scatter(acc_v, [bk, d_vec], xg * exv)
      return carry
    lax.fori_loop(0, EPT // L, _chunk, 0, unroll=4)

    for k in range(FS):
      pltpu.sync_copy(acc_v.at[k], num_out.at[s * FS + k, pl.ds(n0, N)])


def _msg_phase(src_f, dst_f, ex_all, xw_t):
  mesh = plsc.VectorSubcoreMesh(core_axis_name="c", subcore_axis_name="s")
  kernel_fn = functools.partial(
      pl.kernel,
      out_type=jax.ShapeDtypeStruct((HID, NW * N), _f32),
      mesh=mesh,
      scratch_types=[
          pltpu.VMEM((EPT,), _i32),
          pltpu.VMEM((EPT,), _i32),
          pltpu.VMEM((EPT,), _f32),
          pltpu.VMEM((FS, N), _f32),
          pltpu.VMEM((FS, N), _f32),
      ],
      compiler_params=pltpu.CompilerParams(needs_layout_passes=False,
                                           use_tc_tiling_on_sc=False),
  )(_msg_body)
  return kernel_fn(src_f, dst_f, ex_all, xw_t)


def kernel(h, edge_index, W1, a_src1, a_dst1, b1, W2, a_src2, a_dst2, b2,
           W_out, b_out):
  src = edge_index[0].astype(_i32)
  dst = edge_index[1].astype(_i32)
  pad = jnp.zeros((E_PAD - E,), _i32)
  src_f = jnp.concatenate([src, pad])
  dst_f = jnp.concatenate([dst, pad])

  asrc1 = a_src1.reshape(1, HID)
  adst1 = a_dst1.reshape(1, HID)
  asrc2 = a_src2.reshape(1, HID)
  adst2 = a_dst2.reshape(1, HID)

  def layer_edge(xw_t, as_row, ad_row):
    ex, denp = _attn_phase(src_f, dst_f, as_row.reshape(N), ad_row.reshape(N))
    nump = _msg_phase(src_f, dst_f, ex, xw_t)
    return nump.reshape(HID, NW, N), denp.reshape(NW, N)

  xw1, as1, ad1 = _dense_in(h, W1, asrc1, adst1)
  num1, den1 = layer_edge(xw1, as1, ad1)
  xw2, as2, ad2 = _dense_mid(num1, den1, b1.reshape(HID, 1),
                             W2, asrc2, adst2)
  num2, den2 = layer_edge(xw2, as2, ad2)
  out_t = _dense_out(num2, den2, b2.reshape(HID, 1), W_out,
                     b_out.reshape(1, 1))
  return out_t.reshape(N, 1)


# speedup vs baseline: 1.0221x; 1.0221x over previous
import functools

import jax
import jax.numpy as jnp
from jax import lax
from jax.experimental import pallas as pl
from jax.experimental.pallas import tpu as pltpu
from jax.experimental.pallas import tpu_sc as plsc

N = 10000
E = 320000
D_IN = 128
HID = 16
FS = 4

NC = 2
NS = 16
NW = NC * NS
L = 16

EPT = 10240
E_PAD = NW * EPT

_f32 = jnp.float32
_i32 = jnp.int32


def _t_matmul(w_ref, x_ref):
  return lax.dot_general(w_ref[:], x_ref[:], (((0,), (0,)), ((), ())),
                         preferred_element_type=_f32)


def _dense_in_body(h_ref, w_ref, asrc_ref, adst_ref, xw_out, as_out, ad_out):
  xw_t = lax.dot_general(w_ref[:], h_ref[:], (((0,), (1,)), ((), ())),
                         preferred_element_type=_f32)
  xw_out[:] = xw_t
  as_out[:] = jnp.dot(asrc_ref[:], xw_t, preferred_element_type=_f32)
  ad_out[:] = jnp.dot(adst_ref[:], xw_t, preferred_element_type=_f32)


def _dense_in(h, w, asrc, adst):
  return pl.pallas_call(
      _dense_in_body,
      out_shape=[
          jax.ShapeDtypeStruct((HID, N), _f32),
          jax.ShapeDtypeStruct((1, N), _f32),
          jax.ShapeDtypeStruct((1, N), _f32),
      ],
  )(h, w, asrc, adst)


def _combine(nump_ref, denp_ref, b_ref):
  num = jnp.sum(nump_ref[:], axis=1)
  den = jnp.sum(denp_ref[:], axis=0)
  return jnp.maximum(num / (den[None, :] + 1e-16) + b_ref[:], 0.0)


def _dense_mid_body(nump_ref, denp_ref, b_ref, w_ref, asrc_ref, adst_ref,
                    xw_out, as_out, ad_out):
  x_t = _combine(nump_ref, denp_ref, b_ref)
  xw_t = _t_matmul(w_ref, x_t)
  xw_out[:] = xw_t
  as_out[:] = jnp.dot(asrc_ref[:], xw_t, preferred_element_type=_f32)
  ad_out[:] = jnp.dot(adst_ref[:], xw_t, preferred_element_type=_f32)


def _dense_mid(nump, denp, b, w, asrc, adst):
  return pl.pallas_call(
      _dense_mid_body,
      out_shape=[
          jax.ShapeDtypeStruct((HID, N), _f32),
          jax.ShapeDtypeStruct((1, N), _f32),
          jax.ShapeDtypeStruct((1, N), _f32),
      ],
  )(nump, denp, b, w, asrc, adst)


def _dense_out_body(nump_ref, denp_ref, b_ref, w_ref, bout_ref, out_ref):
  x_t = _combine(nump_ref, denp_ref, b_ref)
  out_ref[:] = _t_matmul(w_ref, x_t) + bout_ref[:]


def _dense_out(nump, denp, b, w, bout):
  return pl.pallas_call(
      _dense_out_body,
      out_shape=jax.ShapeDtypeStruct((1, N), _f32),
  )(nump, denp, b, w, bout)


def _edge_body(src_hbm, dst_hbm, asrc_hbm, adst_hbm, xwt_hbm,
               num_out, den_out,
               src_v, dst_v, ex_v, xw_v, acc_v):
  cid = lax.axis_index("c")
  sid = lax.axis_index("s")
  wid = sid * NC + cid
  e0 = pl.multiple_of(wid * EPT, 8)

  pltpu.sync_copy(src_hbm.at[pl.ds(e0, EPT)], src_v)
  pltpu.sync_copy(dst_hbm.at[pl.ds(e0, EPT)], dst_v)
  pltpu.sync_copy(asrc_hbm, xw_v.at[0])
  pltpu.sync_copy(adst_hbm, xw_v.at[1])

  zero16 = jnp.zeros((L,), _f32)

  def _zden(i, carry):
    xw_v[2, pl.ds(i * L, L)] = zero16
    return carry
  lax.fori_loop(0, N // L, _zden, 0)

  iota = lax.iota(_i32, L)
  ebase = wid * EPT
  b0 = jnp.full((L,), 0, _i32)
  b1 = jnp.full((L,), 1, _i32)
  b2 = jnp.full((L,), 2, _i32)

  def _attn_chunk(j, carry):
    s_vec = src_v[pl.ds(j * L, L)]
    d_vec = dst_v[pl.ds(j * L, L)]
    a = (plsc.load_gather(xw_v, [b0, s_vec]) +
         plsc.load_gather(xw_v, [b1, d_vec]))
    a = jnp.maximum(a, a * 0.2)
    ex = jnp.exp(a)
    ex = jnp.where(ebase + j * L + iota < E, ex, 0.0)
    plsc.addupdate_scatter(xw_v, [b2, d_vec], ex)
    ex_v[pl.ds(j * L, L)] = ex
    return carry
  lax.fori_loop(0, EPT // L, _attn_chunk, 0, unroll=4)

  d0 = pl.multiple_of(wid * N, 8)
  pltpu.sync_copy(xw_v.at[2], den_out.at[pl.ds(d0, N)])

  n0 = pl.multiple_of(wid * N, 8)
  for s in range(HID // FS):
    pltpu.sync_copy(xwt_hbm.at[pl.ds(s * FS, FS)], xw_v)

    def _zacc(i, carry):
      acc_v[i % FS, pl.ds((i // FS) * L, L)] = zero16
      return carry
    lax.fori_loop(0, FS * (N // L), _zacc, 0)

    def _msg_chunk(j, carry):
      s_vec = src_v[pl.ds(j * L, L)]
      d_vec = dst_v[pl.ds(j * L, L)]
      exv = ex_v[pl.ds(j * L, L)]
      for k in range(FS):
        bk = jnp.full((L,), k, _i32)
        xg = plsc.load_gather(xw_v, [bk, s_vec])
        plsc.addupdate_scatter(acc_v, [bk, d_vec], xg * exv)
      return carry
    lax.fori_loop(0, EPT // L, _msg_chunk, 0, unroll=4)

    for k in range(FS):
      pltpu.sync_copy(acc_v.at[k], num_out.at[s * FS + k, pl.ds(n0, N)])


def _edge_phase(src_f, dst_f, alpha_src, alpha_dst, xw_t):
  mesh = plsc.VectorSubcoreMesh(core_axis_name="c", subcore_axis_name="s")
  kernel_fn = functools.partial(
      pl.kernel,
      out_type=[
          jax.ShapeDtypeStruct((HID, NW * N), _f32),
          jax.ShapeDtypeStruct((NW * N,), _f32),
      ],
      mesh=mesh,
      scratch_types=[
          pltpu.VMEM((EPT,), _i32),
          pltpu.VMEM((EPT,), _i32),
          pltpu.VMEM((EPT,), _f32),
          pltpu.VMEM((FS, N), _f32),
          pltpu.VMEM((FS, N), _f32),
      ],
      compiler_params=pltpu.CompilerParams(needs_layout_passes=False,
                                           use_tc_tiling_on_sc=False),
  )(_edge_body)
  return kernel_fn(src_f, dst_f, alpha_src, alpha_dst, xw_t)


def kernel(h, edge_index, W1, a_src1, a_dst1, b1, W2, a_src2, a_dst2, b2,
           W_out, b_out):
  src = edge_index[0].astype(_i32)
  dst = edge_index[1].astype(_i32)
  pad = jnp.zeros((E_PAD - E,), _i32)
  src_f = jnp.concatenate([src, pad])
  dst_f = jnp.concatenate([dst, pad])

  asrc1 = a_src1.reshape(1, HID)
  adst1 = a_dst1.reshape(1, HID)
  asrc2 = a_src2.reshape(1, HID)
  adst2 = a_dst2.reshape(1, HID)

  def layer_edge(xw_t, as_row, ad_row):
    nump, denp = _edge_phase(src_f, dst_f, as_row.reshape(N),
                             ad_row.reshape(N), xw_t)
    return nump.reshape(HID, NW, N), denp.reshape(NW, N)

  xw1, as1, ad1 = _dense_in(h, W1, asrc1, adst1)
  num1, den1 = layer_edge(xw1, as1, ad1)
  xw2, as2, ad2 = _dense_mid(num1, den1, b1.reshape(HID, 1),
                             W2, asrc2, adst2)
  num2, den2 = layer_edge(xw2, as2, ad2)
  out_t = _dense_out(num2, den2, b2.reshape(HID, 1), W_out,
                     b_out.reshape(1, 1))
  return out_t.reshape(N, 1)
